# Initial kernel scaffold; baseline (speedup 1.0000x reference)
import jax
import jax.numpy as jnp
from jax.experimental import pallas as pl


def kernel(x, conv1_w, conv2_w, conv3_w, conv4_w, conv5_w, bn1_g, bn1_b, bn2_g, bn2_b, bn3_g, bn3_b, bn4_g, bn4_b, bn5_g, bn5_b, lin1_w, bn6_g, bn6_b, lin2_w, lin2_b, bn7_g, bn7_b, lin3_w, lin3_b):
    raise NotImplementedError("write your pallas kernel here")


# speedup vs baseline: 7.2111x; 7.2111x over previous
import functools

import jax
import jax.numpy as jnp
from jax import lax
from jax.experimental import pallas as pl
from jax.experimental.pallas import tpu as pltpu
from jax.experimental.pallas import tpu_sc as plsc

EPS = 1e-5
K = 20
B = 16
N = 1024
TILE = 128
NTILES = N // TILE
NEG = -3.0e38


def _leaky(v):
    return jnp.where(v >= 0, v, 0.2 * v)


def _layer_kernel(xf_ref, xr_ref, sqr_ref, sqc_ref, w_ref, idx_ref, u_ref):
    b = pl.program_id(0)
    x_full = xf_ref[0]
    x_rows = xr_ref[0]
    dn = (((1,), (1,)), ((), ()))
    gram = lax.dot_general(x_rows.astype(jnp.bfloat16),
                           x_full.astype(jnp.bfloat16), dn,
                           preferred_element_type=jnp.float32)
    inner = -2.0 * gram
    score = ((-sqr_ref[0]) - inner) - sqc_ref[0]

    col_iota = lax.broadcasted_iota(jnp.int32, (TILE, N), 1)
    k_iota = lax.broadcasted_iota(jnp.int32, (TILE, K), 1)

    def body(t, carry):
        sc, acc = carry
        mx = jnp.max(sc, axis=1, keepdims=True)
        am = jnp.min(jnp.where(sc == mx, col_iota, jnp.int32(N)),
                     axis=1, keepdims=True)
        acc = jnp.where(k_iota == t, am, acc)
        sc = jnp.where(col_iota == am, NEG, sc)
        return sc, acc

    _, idxacc = lax.fori_loop(0, K, body,
                              (score, jnp.zeros((TILE, K), jnp.int32)))
    idx_ref[0] = idxacc + b * N

    u = lax.dot_general(x_rows.astype(jnp.bfloat16),
                        w_ref[...].astype(jnp.bfloat16),
                        (((1,), (0,)), ((), ())),
                        preferred_element_type=jnp.float32)
    u_ref[0] = u


def _layer_call(xl, sqr, sqc, wT, cout):
    return pl.pallas_call(
        _layer_kernel,
        grid=(B, NTILES),
        in_specs=[
            pl.BlockSpec((1, N, 128), lambda b, r: (b, 0, 0)),
            pl.BlockSpec((1, TILE, 128), lambda b, r: (b, r, 0)),
            pl.BlockSpec((1, TILE, 1), lambda b, r: (b, r, 0)),
            pl.BlockSpec((1, 1, N), lambda b, r: (b, 0, 0)),
            pl.BlockSpec((128, cout), lambda b, r: (0, 0)),
        ],
        out_specs=[
            pl.BlockSpec((1, TILE, K), lambda b, r: (b, r, 0)),
            pl.BlockSpec((1, TILE, cout), lambda b, r: (b, r, 0)),
        ],
        out_shape=[
            jax.ShapeDtypeStruct((B, N, K), jnp.int32),
            jax.ShapeDtypeStruct((B, N, cout), jnp.float32),
        ],
    )(xl, xl, sqr, sqc, wT)


_P = 32
_IDXROWS = _P * K // 128


def _neighbor_max_sc(u_flat, idx2d, c):
    nw = 32
    pts_per_w = (B * N) // nw
    chunks = pts_per_w // _P
    mesh = plsc.VectorSubcoreMesh(core_axis_name="c", subcore_axis_name="s")

    @functools.partial(
        pl.kernel,
        out_type=jax.ShapeDtypeStruct((B * N, c), jnp.float32),
        mesh=mesh,
        compiler_params=pltpu.CompilerParams(use_tc_tiling_on_sc=False),
        scratch_types=[
            pltpu.VMEM((pts_per_w * K // 128, 128), jnp.int32),
            pltpu.VMEM((_P * K, c), jnp.float32),
            pltpu.VMEM((_P, c), jnp.float32),
            pltpu.SemaphoreType.DMA,
        ],
    )
    def kern(u_hbm, i_hbm, o_hbm, idx_v, rows_v, out_v, sem):
        wid = lax.axis_index("s") * 2 + lax.axis_index("c")
        base = wid * pts_per_w
        nidxrows = pts_per_w * K // 128
        pltpu.sync_copy(i_hbm.at[pl.ds(wid * nidxrows, nidxrows)], idx_v)

        @pl.loop(0, chunks)
        def _(ci):
            pbase = base + ci * _P
            copies = []
            for ri in range(_IDXROWS):
                copies.append(pltpu.async_copy(
                    u_hbm.at[idx_v.at[ci * _IDXROWS + ri]],
                    rows_v.at[pl.ds(ri * 128, 128)], sem))
            for cp in copies:
                cp.wait()

            @pl.loop(0, _P)
            def _(p):
                for cc in range(c // 16):
                    sl = pl.ds(cc * 16, 16)
                    acc = rows_v[p * K, sl]
                    for j in range(1, K):
                        acc = jnp.maximum(acc, rows_v[p * K + j, sl])
                    out_v[p, sl] = acc

            pltpu.sync_copy(out_v, o_hbm.at[pl.ds(pbase, _P)])

    return kern(u_flat, idx2d)


def _gather_max(u, idx, c):
    u_flat = u.reshape(B * N, c)
    idx2d = idx.reshape(B * N * K // 128, 128)
    return _neighbor_max_sc(u_flat, idx2d, c).reshape(B, N, c)


def _head1_kernel(xc_ref, w5T_ref, y_ref):
    y = lax.dot_general(xc_ref[0].astype(jnp.bfloat16),
                        w5T_ref[...].astype(jnp.bfloat16),
                        (((1,), (0,)), ((), ())),
                        preferred_element_type=jnp.float32)
    y_ref[0] = y


def _head1_call(xc, w5T):
    return pl.pallas_call(
        _head1_kernel,
        grid=(B,),
        in_specs=[
            pl.BlockSpec((1, N, 640), lambda b: (b, 0, 0)),
            pl.BlockSpec((640, 1024), lambda b: (0, 0)),
        ],
        out_specs=pl.BlockSpec((1, N, 1024), lambda b: (b, 0, 0)),
        out_shape=jax.ShapeDtypeStruct((B, N, 1024), jnp.float32),
    )(xc, w5T)


def _head2_kernel(z_ref, out_ref):
    out_ref[0] = jnp.max(z_ref[0], axis=1, keepdims=True)


def _head2_call(z):
    return pl.pallas_call(
        _head2_kernel,
        grid=(B,),
        in_specs=[pl.BlockSpec((1, N, 1024), lambda b: (b, 0, 0))],
        out_specs=pl.BlockSpec((1, N, 1), lambda b: (b, 0, 0)),
        out_shape=jax.ShapeDtypeStruct((B, N, 1), jnp.float32),
    )(z)


def _bn1d(h, g, bta):
    mu = jnp.mean(h, axis=0, keepdims=True)
    va = jnp.mean((h - mu) * (h - mu), axis=0, keepdims=True)
    return g * (h - mu) / jnp.sqrt(va + EPS) + bta


def _head3_kernel(p_ref, w1_ref, gb6_ref, w2_ref, b2_ref, gb7_ref,
                  w3_ref, b3_ref, out_ref):
    dn = (((1,), (0,)), ((), ()))

    def mm(a, w_):
        return lax.dot_general(a.astype(jnp.bfloat16),
                               w_.astype(jnp.bfloat16), dn,
                               preferred_element_type=jnp.float32)

    h = mm(p_ref[...], w1_ref[...])
    h = _leaky(_bn1d(h, gb6_ref[0:1, :], gb6_ref[1:2, :]))
    h = mm(h, w2_ref[...]) + b2_ref[0:1, :]
    h = _leaky(_bn1d(h, gb7_ref[0:1, :], gb7_ref[1:2, :]))
    h = mm(h, w3_ref[...]) + b3_ref[0:1, :]
    out_ref[...] = h


def _head3_call(pooled, lin1T, gb6, lin2T, b2, gb7, lin3T, b3):
    fullspec = lambda shp: pl.BlockSpec(shp, lambda: (0,) * len(shp))
    return pl.pallas_call(
        _head3_kernel,
        grid=(),
        in_specs=[fullspec((16, 1024)), fullspec((1024, 512)),
                  fullspec((8, 512)), fullspec((512, 256)), fullspec((8, 256)),
                  fullspec((8, 256)), fullspec((256, 40)), fullspec((8, 40))],
        out_specs=fullspec((16, 40)),
        out_shape=jax.ShapeDtypeStruct((16, 40), jnp.float32),
    )(pooled, lin1T, gb6, lin2T, b2, gb7, lin3T, b3)


def _bn_leaky_full(u, g, bta):
    u4 = lax.optimization_barrier(u.transpose(0, 2, 1)[:, :, :, None])
    m = jnp.mean(u4, axis=(0, 2, 3), keepdims=True)
    v = jnp.var(u4, axis=(0, 2, 3), keepdims=True)
    h4 = _leaky(g.reshape(1, -1, 1, 1) * (u4 - m) / jnp.sqrt(v + EPS)
                + bta.reshape(1, -1, 1, 1))
    h = h4.reshape(B, -1, N).transpose(0, 2, 1)
    return h, m.reshape(-1), v.reshape(-1)


def _conv_bn_leaky_ref(xprev4, w, g, bta):
    c4 = lax.optimization_barrier(jnp.einsum('oc,bcnk->bonk', w, xprev4))
    m = jnp.mean(c4, axis=(0, 2, 3), keepdims=True)
    v = jnp.var(c4, axis=(0, 2, 3), keepdims=True)
    h4 = _leaky(g.reshape(1, -1, 1, 1) * (c4 - m) / jnp.sqrt(v + EPS)
                + bta.reshape(1, -1, 1, 1))
    return h4, m, v


def _edge(xprev4, w, mgath, g, bta):
    h4, mean, var = _conv_bn_leaky_ref(xprev4, w, g, bta)
    m4 = mgath.transpose(0, 2, 1)[:, :, :, None]
    hm4 = _leaky(g.reshape(1, -1, 1, 1) * (m4 - mean) / jnp.sqrt(var + EPS)
                 + bta.reshape(1, -1, 1, 1))
    edge4 = jnp.concatenate([hm4 - h4, h4], axis=1)
    return edge4.reshape(B, -1, N).transpose(0, 2, 1)


def _to_ref_layout(xl):
    return lax.optimization_barrier(xl.transpose(0, 2, 1)[:, :, :, None])


def _sqnorm(xl):
    xlt = xl.transpose(0, 2, 1)
    xx = jnp.sum(xlt * xlt, axis=1, keepdims=True)
    return xx.reshape(B, N, 1), xx


def _pack2(a, bta):
    out = jnp.zeros((8, a.shape[0]), jnp.float32)
    return out.at[0].set(a).at[1].set(bta)


def _b2row(bvec):
    out = jnp.zeros((8, bvec.shape[0]), jnp.float32)
    return out.at[0].set(bvec)


def kernel(x, conv1_w, conv2_w, conv3_w, conv4_w, conv5_w, bn1_g, bn1_b,
           bn2_g, bn2_b, bn3_g, bn3_b, bn4_g, bn4_b, bn5_g, bn5_b, lin1_w,
           bn6_g, bn6_b, lin2_w, lin2_b, bn7_g, bn7_b, lin3_w, lin3_b):
    xp = jnp.pad(x.transpose(0, 2, 1), ((0, 0), (0, 0), (0, 125)))
    w1T = jnp.pad(conv1_w.T, ((0, 125), (0, 0)))
    sqc = jnp.sum(x * x, axis=1, keepdims=True)
    sqr = sqc.reshape(B, N, 1)

    idx1, u1 = _layer_call(xp, sqr, sqc, w1T, 64)
    m1 = _gather_max(u1, idx1, 64)
    x1 = _edge(x.reshape(B, 3, N, 1), conv1_w, m1, bn1_g, bn1_b)

    sqr, sqc = _sqnorm(x1)
    idx2, u2 = _layer_call(x1, sqr, sqc, conv2_w.T, 64)
    m2 = _gather_max(u2, idx2, 64)
    x2 = _edge(_to_ref_layout(x1), conv2_w, m2, bn2_g, bn2_b)

    sqr, sqc = _sqnorm(x2)
    idx3, u3 = _layer_call(x2, sqr, sqc, conv3_w.T, 64)
    m3 = _gather_max(u3, idx3, 64)
    x3 = _edge(_to_ref_layout(x2), conv3_w, m3, bn3_g, bn3_b)

    sqr, sqc = _sqnorm(x3)
    idx4, u4 = _layer_call(x3, sqr, sqc, conv4_w.T, 128)
    m4 = _gather_max(u4, idx4, 128)
    x4 = _edge(_to_ref_layout(x3), conv4_w, m4, bn4_g, bn4_b)

    xc = jnp.concatenate([x1, x2, x3, x4], axis=-1)
    y = _head1_call(xc, conv5_w.T)
    z, _, _ = _bn_leaky_full(y, bn5_g, bn5_b)
    pooled = _head2_call(z).reshape(B, N)

    return _head3_call(pooled, lin1_w.T, _pack2(bn6_g, bn6_b), lin2_w.T,
                       _b2row(lin2_b), _pack2(bn7_g, bn7_b), lin3_w.T,
                       _b2row(lin3_b))


# speedup vs baseline: 7.2340x; 1.0032x over previous
import functools

import jax
import jax.numpy as jnp
from jax import lax
from jax.experimental import pallas as pl
from jax.experimental.pallas import tpu as pltpu
from jax.experimental.pallas import tpu_sc as plsc

EPS = 1e-5
K = 20
B = 16
N = 1024
TILE = 128
NTILES = N // TILE
NEG = -3.0e38


def _leaky(v):
    return jnp.where(v >= 0, v, 0.2 * v)


def _layer_kernel(xf_ref, xr_ref, sqr_ref, sqc_ref, w_ref, idx_ref, u_ref):
    b = pl.program_id(0)
    x_full = xf_ref[0]
    x_rows = xr_ref[0]
    dn = (((1,), (1,)), ((), ()))
    gram = lax.dot_general(x_rows.astype(jnp.bfloat16),
                           x_full.astype(jnp.bfloat16), dn,
                           preferred_element_type=jnp.float32)
    inner = -2.0 * gram
    score = ((-sqr_ref[0]) - inner) - sqc_ref[0]

    col_iota = lax.broadcasted_iota(jnp.int32, (TILE, N), 1)
    k_iota = lax.broadcasted_iota(jnp.int32, (TILE, K), 1)

    def body(t, carry):
        sc, acc = carry
        mx = jnp.max(sc, axis=1, keepdims=True)
        am = jnp.min(jnp.where(sc == mx, col_iota, jnp.int32(N)),
                     axis=1, keepdims=True)
        acc = jnp.where(k_iota == t, am, acc)
        sc = jnp.where(col_iota == am, NEG, sc)
        return sc, acc

    _, idxacc = lax.fori_loop(0, K, body,
                              (score, jnp.zeros((TILE, K), jnp.int32)))
    idx_ref[0] = idxacc + b * N

    u = lax.dot_general(x_rows.astype(jnp.bfloat16),
                        w_ref[...].astype(jnp.bfloat16),
                        (((1,), (0,)), ((), ())),
                        preferred_element_type=jnp.float32)
    u_ref[0] = u


def _layer_call(xl, sqr, sqc, wT, cout):
    return pl.pallas_call(
        _layer_kernel,
        grid=(B, NTILES),
        in_specs=[
            pl.BlockSpec((1, N, 128), lambda b, r: (b, 0, 0)),
            pl.BlockSpec((1, TILE, 128), lambda b, r: (b, r, 0)),
            pl.BlockSpec((1, TILE, 1), lambda b, r: (b, r, 0)),
            pl.BlockSpec((1, 1, N), lambda b, r: (b, 0, 0)),
            pl.BlockSpec((128, cout), lambda b, r: (0, 0)),
        ],
        out_specs=[
            pl.BlockSpec((1, TILE, K), lambda b, r: (b, r, 0)),
            pl.BlockSpec((1, TILE, cout), lambda b, r: (b, r, 0)),
        ],
        out_shape=[
            jax.ShapeDtypeStruct((B, N, K), jnp.int32),
            jax.ShapeDtypeStruct((B, N, cout), jnp.float32),
        ],
    )(xl, xl, sqr, sqc, wT)


_P = 32
_IDXROWS = _P * K // 128


def _neighbor_max_sc(u_flat, idx2d, c):
    nw = 32
    pts_per_w = (B * N) // nw
    chunks = pts_per_w // _P
    mesh = plsc.VectorSubcoreMesh(core_axis_name="c", subcore_axis_name="s")

    @functools.partial(
        pl.kernel,
        out_type=jax.ShapeDtypeStruct((B * N, c), jnp.float32),
        mesh=mesh,
        compiler_params=pltpu.CompilerParams(use_tc_tiling_on_sc=False),
        scratch_types=[
            pltpu.VMEM((pts_per_w * K // 128, 128), jnp.int32),
            pltpu.VMEM((_P * K, c), jnp.float32),
            pltpu.VMEM((_P, c), jnp.float32),
            pltpu.SemaphoreType.DMA,
        ],
    )
    def kern(u_hbm, i_hbm, o_hbm, idx_v, rows_v, out_v, sem):
        wid = lax.axis_index("s") * 2 + lax.axis_index("c")
        base = wid * pts_per_w
        nidxrows = pts_per_w * K // 128
        pltpu.sync_copy(i_hbm.at[pl.ds(wid * nidxrows, nidxrows)], idx_v)

        @pl.loop(0, chunks)
        def _(ci):
            pbase = base + ci * _P
            copies = []
            for ri in range(_IDXROWS):
                copies.append(pltpu.async_copy(
                    u_hbm.at[idx_v.at[ci * _IDXROWS + ri]],
                    rows_v.at[pl.ds(ri * 128, 128)], sem))
            for cp in copies:
                cp.wait()

            @pl.loop(0, _P)
            def _(p):
                for cc in range(c // 16):
                    sl = pl.ds(cc * 16, 16)
                    acc = rows_v[p * K, sl]
                    for j in range(1, K):
                        acc = jnp.maximum(acc, rows_v[p * K + j, sl])
                    out_v[p, sl] = acc

            pltpu.sync_copy(out_v, o_hbm.at[pl.ds(pbase, _P)])

    return kern(u_flat, idx2d)


def _gather_max(u, idx, c):
    u_flat = u.reshape(B * N, c)
    idx2d = idx.reshape(B * N * K // 128, 128)
    return _neighbor_max_sc(u_flat, idx2d, c).reshape(B, N, c)


def _head1_kernel(xc_ref, w5T_ref, y_ref):
    y = lax.dot_general(xc_ref[0].astype(jnp.bfloat16),
                        w5T_ref[...].astype(jnp.bfloat16),
                        (((1,), (0,)), ((), ())),
                        preferred_element_type=jnp.float32)
    y_ref[0] = y


def _head1_call(xc, w5T):
    return pl.pallas_call(
        _head1_kernel,
        grid=(B,),
        in_specs=[
            pl.BlockSpec((1, N, 640), lambda b: (b, 0, 0)),
            pl.BlockSpec((640, 1024), lambda b: (0, 0)),
        ],
        out_specs=pl.BlockSpec((1, N, 1024), lambda b: (b, 0, 0)),
        out_shape=jax.ShapeDtypeStruct((B, N, 1024), jnp.float32),
    )(xc, w5T)


def _head2_kernel(z_ref, out_ref):
    out_ref[0] = jnp.max(z_ref[0], axis=1, keepdims=True)


def _head2_call(z):
    return pl.pallas_call(
        _head2_kernel,
        grid=(B,),
        in_specs=[pl.BlockSpec((1, N, 1024), lambda b: (b, 0, 0))],
        out_specs=pl.BlockSpec((1, N, 1), lambda b: (b, 0, 0)),
        out_shape=jax.ShapeDtypeStruct((B, N, 1), jnp.float32),
    )(z)


def _bn1d(h, g, bta):
    mu = jnp.mean(h, axis=0, keepdims=True)
    va = jnp.mean((h - mu) * (h - mu), axis=0, keepdims=True)
    return g * (h - mu) / jnp.sqrt(va + EPS) + bta


def _head3_kernel(p_ref, w1_ref, gb6_ref, w2_ref, b2_ref, gb7_ref,
                  w3_ref, b3_ref, out_ref):
    dn = (((1,), (0,)), ((), ()))

    def mm(a, w_):
        return lax.dot_general(a.astype(jnp.bfloat16),
                               w_.astype(jnp.bfloat16), dn,
                               preferred_element_type=jnp.float32)

    h = mm(p_ref[...], w1_ref[...])
    h = _leaky(_bn1d(h, gb6_ref[0:1, :], gb6_ref[1:2, :]))
    h = mm(h, w2_ref[...]) + b2_ref[0:1, :]
    h = _leaky(_bn1d(h, gb7_ref[0:1, :], gb7_ref[1:2, :]))
    h = mm(h, w3_ref[...]) + b3_ref[0:1, :]
    out_ref[...] = h


def _head3_call(pooled, lin1T, gb6, lin2T, b2, gb7, lin3T, b3):
    fullspec = lambda shp: pl.BlockSpec(shp, lambda: (0,) * len(shp))
    return pl.pallas_call(
        _head3_kernel,
        grid=(),
        in_specs=[fullspec((16, 1024)), fullspec((1024, 512)),
                  fullspec((8, 512)), fullspec((512, 256)), fullspec((8, 256)),
                  fullspec((8, 256)), fullspec((256, 40)), fullspec((8, 40))],
        out_specs=fullspec((16, 40)),
        out_shape=jax.ShapeDtypeStruct((16, 40), jnp.float32),
    )(pooled, lin1T, gb6, lin2T, b2, gb7, lin3T, b3)


def _bn_leaky_full(u, g, bta):
    u4 = lax.optimization_barrier(u.transpose(0, 2, 1)[:, :, :, None])
    m = jnp.mean(u4, axis=(0, 2, 3), keepdims=True)
    v = jnp.var(u4, axis=(0, 2, 3), keepdims=True)
    h4 = _leaky(g.reshape(1, -1, 1, 1) * (u4 - m) / jnp.sqrt(v + EPS)
                + bta.reshape(1, -1, 1, 1))
    h = h4.reshape(B, -1, N).transpose(0, 2, 1)
    return h, m.reshape(-1), v.reshape(-1)


def _conv_bn_leaky_ref(xprev4, w, g, bta):
    c4 = jnp.einsum('oc,bcnk->bonk', w, xprev4)
    m = jnp.mean(c4, axis=(0, 2, 3), keepdims=True)
    v = jnp.var(c4, axis=(0, 2, 3), keepdims=True)
    h4 = _leaky(g.reshape(1, -1, 1, 1) * (c4 - m) / jnp.sqrt(v + EPS)
                + bta.reshape(1, -1, 1, 1))
    return h4, m, v


def _edge(xprev4, w, mgath, g, bta):
    h4, mean, var = _conv_bn_leaky_ref(xprev4, w, g, bta)
    m4 = mgath.transpose(0, 2, 1)[:, :, :, None]
    hm4 = _leaky(g.reshape(1, -1, 1, 1) * (m4 - mean) / jnp.sqrt(var + EPS)
                 + bta.reshape(1, -1, 1, 1))
    edge4 = jnp.concatenate([hm4 - h4, h4], axis=1)
    return edge4.reshape(B, -1, N).transpose(0, 2, 1)


def _to_ref_layout(xl):
    return lax.optimization_barrier(xl.transpose(0, 2, 1)[:, :, :, None])


def _sqnorm(xl):
    xlt = xl.transpose(0, 2, 1)
    xx = jnp.sum(xlt * xlt, axis=1, keepdims=True)
    return xx.reshape(B, N, 1), xx


def _pack2(a, bta):
    out = jnp.zeros((8, a.shape[0]), jnp.float32)
    return out.at[0].set(a).at[1].set(bta)


def _b2row(bvec):
    out = jnp.zeros((8, bvec.shape[0]), jnp.float32)
    return out.at[0].set(bvec)


def kernel(x, conv1_w, conv2_w, conv3_w, conv4_w, conv5_w, bn1_g, bn1_b,
           bn2_g, bn2_b, bn3_g, bn3_b, bn4_g, bn4_b, bn5_g, bn5_b, lin1_w,
           bn6_g, bn6_b, lin2_w, lin2_b, bn7_g, bn7_b, lin3_w, lin3_b):
    xp = jnp.pad(x.transpose(0, 2, 1), ((0, 0), (0, 0), (0, 125)))
    w1T = jnp.pad(conv1_w.T, ((0, 125), (0, 0)))
    sqc = jnp.sum(x * x, axis=1, keepdims=True)
    sqr = sqc.reshape(B, N, 1)

    idx1, u1 = _layer_call(xp, sqr, sqc, w1T, 64)
    m1 = _gather_max(u1, idx1, 64)
    x1 = _edge(lax.optimization_barrier(x).reshape(B, 3, N, 1), conv1_w,
               m1, bn1_g, bn1_b)

    sqr, sqc = _sqnorm(x1)
    idx2, u2 = _layer_call(x1, sqr, sqc, conv2_w.T, 64)
    m2 = _gather_max(u2, idx2, 64)
    x2 = _edge(_to_ref_layout(x1), conv2_w, m2, bn2_g, bn2_b)

    sqr, sqc = _sqnorm(x2)
    idx3, u3 = _layer_call(x2, sqr, sqc, conv3_w.T, 64)
    m3 = _gather_max(u3, idx3, 64)
    x3 = _edge(_to_ref_layout(x2), conv3_w, m3, bn3_g, bn3_b)

    sqr, sqc = _sqnorm(x3)
    idx4, u4 = _layer_call(x3, sqr, sqc, conv4_w.T, 128)
    m4 = _gather_max(u4, idx4, 128)
    x4 = _edge(_to_ref_layout(x3), conv4_w, m4, bn4_g, bn4_b)

    xc = jnp.concatenate([x1, x2, x3, x4], axis=-1)
    y = _head1_call(xc, conv5_w.T)
    z, _, _ = _bn_leaky_full(y, bn5_g, bn5_b)
    pooled = _head2_call(z).reshape(B, N)

    return _head3_call(pooled, lin1_w.T, _pack2(bn6_g, bn6_b), lin2_w.T,
                       _b2row(lin2_b), _pack2(bn7_g, bn7_b), lin3_w.T,
                       _b2row(lin3_b))
